# Initial kernel scaffold; baseline (speedup 1.0000x reference)
import jax
import jax.numpy as jnp
from jax.experimental import pallas as pl


def kernel(x, edge_index, W1_l, b1, W1_r, W2_l, b2, W2_r):
    raise NotImplementedError("write your pallas kernel here")


# speedup vs baseline: 5.2786x; 5.2786x over previous
import functools

import jax
import jax.numpy as jnp
from jax import lax
from jax.experimental import pallas as pl
from jax.experimental.pallas import tpu as pltpu
from jax.experimental.pallas import tpu_sc as plsc

N = 10000
E = 320000
D = 128

NC = 2
NS = 16
NW = NC * NS
EPW = E // NW
CH = 128
NFULL = EPW // CH
REM = EPW - NFULL * CH
RPT = 632
NP = RPT * NS
CW = 16


def _sc_agg_body(x_hbm, src_hbm, dst_hbm, z_hbm, sum_hbm,
                 src_v, dst_v, rows_v, src_r, dst_r, rows_r, acc_sh, sem):
    c = lax.axis_index("c")
    s = lax.axis_index("s")
    wid = c * NS + s
    base = wid * EPW
    r0 = pl.multiple_of(s * RPT, 8)

    pltpu.sync_copy(z_hbm.at[pl.ds(r0, RPT)], acc_sh.at[pl.ds(r0, RPT)])
    plsc.subcore_barrier()

    @pl.loop(0, NFULL)
    def _(j):
        off = base + j * CH
        pltpu.sync_copy(src_hbm.at[pl.ds(off, CH)], src_v)
        pltpu.sync_copy(dst_hbm.at[pl.ds(off, CH)], dst_v)
        pltpu.async_copy(x_hbm.at[src_v], rows_v, sem).wait()
        pltpu.sync_copy(rows_v, acc_sh.at[dst_v], add=True)

    off = base + NFULL * CH
    pltpu.sync_copy(src_hbm.at[pl.ds(off, REM)], src_r)
    pltpu.sync_copy(dst_hbm.at[pl.ds(off, REM)], dst_r)
    pltpu.async_copy(x_hbm.at[src_r], rows_r, sem).wait()
    pltpu.sync_copy(rows_r, acc_sh.at[dst_r], add=True)

    plsc.subcore_barrier()
    pltpu.sync_copy(acc_sh.at[pl.ds(r0, RPT)], sum_hbm.at[c, pl.ds(r0, RPT)])


def _sc_cnt_body(dst_hbm, z_hbm, ones_hbm, cnt_hbm,
                 dst_v, dst_r, ones_v, ones_r, acc_sh, sem):
    del sem
    c = lax.axis_index("c")
    s = lax.axis_index("s")
    wid = c * NS + s
    base = wid * EPW
    r0 = pl.multiple_of(s * RPT, 8)

    pltpu.sync_copy(z_hbm.at[pl.ds(r0, RPT)], acc_sh.at[pl.ds(r0, RPT)])
    pltpu.sync_copy(ones_hbm, ones_v)
    pltpu.sync_copy(ones_hbm.at[pl.ds(0, REM)], ones_r)
    plsc.subcore_barrier()

    @pl.loop(0, NFULL)
    def _(j):
        off = base + j * CH
        pltpu.sync_copy(dst_hbm.at[pl.ds(off, CH)], dst_v)
        pltpu.sync_copy(ones_v, acc_sh.at[dst_v], add=True)

    off = base + NFULL * CH
    pltpu.sync_copy(dst_hbm.at[pl.ds(off, REM)], dst_r)
    pltpu.sync_copy(ones_r, acc_sh.at[dst_r], add=True)

    plsc.subcore_barrier()
    pltpu.sync_copy(acc_sh.at[pl.ds(r0, RPT)], cnt_hbm.at[c, pl.ds(r0, RPT)])


_SC_MESH = plsc.VectorSubcoreMesh(core_axis_name="c", subcore_axis_name="s")

_sc_agg = pl.kernel(
    _sc_agg_body,
    out_type=jax.ShapeDtypeStruct((NC, NP, D), jnp.float32),
    mesh=_SC_MESH,
    scratch_types=[
        pltpu.VMEM((CH,), jnp.int32),
        pltpu.VMEM((CH,), jnp.int32),
        pltpu.VMEM((CH, D), jnp.float32),
        pltpu.VMEM((REM,), jnp.int32),
        pltpu.VMEM((REM,), jnp.int32),
        pltpu.VMEM((REM, D), jnp.float32),
        pltpu.VMEM_SHARED((NP, D), jnp.float32),
        pltpu.SemaphoreType.DMA,
    ],
)

_sc_cnt = pl.kernel(
    _sc_cnt_body,
    out_type=jax.ShapeDtypeStruct((NC, NP, D), jnp.float32),
    mesh=_SC_MESH,
    scratch_types=[
        pltpu.VMEM((CH,), jnp.int32),
        pltpu.VMEM((REM,), jnp.int32),
        pltpu.VMEM((CH, D), jnp.float32),
        pltpu.VMEM((REM, D), jnp.float32),
        pltpu.VMEM_SHARED((NP, D), jnp.float32),
        pltpu.SemaphoreType.DMA,
    ],
)


BLK = 2000


def _tc_body(final, sa, sb, ca, cb, x, wl, wr, b, o):
    cnt = jnp.maximum(ca[:, 0:1] + cb[:, 0:1], 1.0)
    aggr = (sa[:, :] + sb[:, :]) / cnt
    h = (jnp.dot(aggr, wl[:, :], preferred_element_type=jnp.float32)
         + jnp.dot(x[:, :], wr[:, :], preferred_element_type=jnp.float32)
         + b[:, :])
    if final:
        m = jnp.max(h, axis=1, keepdims=True)
        lse = jnp.log(jnp.sum(jnp.exp(h - m), axis=1, keepdims=True)) + m
        o[:, :] = h - lse
    else:
        o[:, :] = jnp.maximum(h, 0.0)


def _make_tc(final):
    row = pl.BlockSpec((BLK, D), lambda i: (i, 0))
    cntspec = pl.BlockSpec((BLK, CW), lambda i: (i, 0))
    full = pl.BlockSpec((D, D), lambda i: (0, 0))
    bias = pl.BlockSpec((1, D), lambda i: (0, 0))
    return pl.pallas_call(
        functools.partial(_tc_body, final),
        grid=(N // BLK,),
        in_specs=[row, row, cntspec, cntspec, row, full, full, bias],
        out_specs=row,
        out_shape=jax.ShapeDtypeStruct((N, D), jnp.float32),
    )


_tc_relu = _make_tc(False)
_tc_logsm = _make_tc(True)


def kernel(x, edge_index, W1_l, b1, W1_r, W2_l, b2, W2_r):
    src = edge_index[0].astype(jnp.int32)
    dst = edge_index[1].astype(jnp.int32)
    zeros = jnp.zeros((NP, D), jnp.float32)
    ones = jnp.ones((CH, D), jnp.float32)

    cnt = _sc_cnt(dst, zeros, ones)
    ca = cnt[0, :N, :CW]
    cb = cnt[1, :N, :CW]
    sum1 = _sc_agg(x, src, dst, zeros)
    h = _tc_relu(sum1[0, :N], sum1[1, :N], ca, cb, x,
                 W1_l.T, W1_r.T, b1[None, :])
    sum2 = _sc_agg(h, src, dst, zeros)
    return _tc_logsm(sum2[0, :N], sum2[1, :N], ca, cb, h,
                     W2_l.T, W2_r.T, b2[None, :])


# speedup vs baseline: 9.1967x; 1.7423x over previous
import functools

import jax
import jax.numpy as jnp
from jax import lax
from jax.experimental import pallas as pl
from jax.experimental.pallas import tpu as pltpu
from jax.experimental.pallas import tpu_sc as plsc

N = 10000
E = 320000
D = 128

NC = 2
NS = 16
NW = NC * NS
EPW = E // NW
CH = 64
NCH = 160
EPWP = NCH * CH
PAD = EPWP - EPW
RPT = 632
NP = RPT * NS
CW = 16
B = 4
QB = 2 * B
GRPC = QB
NGRP = NCH // GRPC - 1
KC = 8


def _sc_agg_body(x_hbm, ip_hbm, z_hbm, sum_hbm,
                 idx_v, rows_v, acc_sh, g0, g1, g2, g3, i0, i1, i2, i3):
    gs = (g0, g1, g2, g3)
    isems = (i0, i1, i2, i3)
    c = lax.axis_index("c")
    s = lax.axis_index("s")
    wid = c * NS + s
    r0 = pl.multiple_of(s * RPT, 8)

    pltpu.sync_copy(z_hbm.at[pl.ds(r0, RPT)], acc_sh.at[pl.ds(r0, RPT)])
    plsc.subcore_barrier()

    for b in range(B):
        pltpu.sync_copy(ip_hbm.at[wid, b], idx_v.at[b])
        pltpu.async_copy(x_hbm.at[idx_v.at[b, 0]], rows_v.at[b], gs[b])

    def step(k, u, issue_next):
        b = u % B
        q = u % QB
        qn = (u + B) % QB
        pltpu.make_async_copy(
            x_hbm.at[idx_v.at[q, 0]], rows_v.at[b], gs[b]).wait()
        if issue_next:
            pltpu.async_copy(ip_hbm.at[wid, k + B], idx_v.at[qn], isems[b])
        pltpu.sync_copy(rows_v.at[b], acc_sh.at[idx_v.at[q, 1]], add=True)
        if issue_next:
            pltpu.make_async_copy(
                ip_hbm.at[wid, k + B], idx_v.at[qn], isems[b]).wait()
            pltpu.async_copy(x_hbm.at[idx_v.at[qn, 0]], rows_v.at[b], gs[b])

    @pl.loop(0, NGRP)
    def _(g):
        for u in range(GRPC):
            step(g * GRPC + u, u, True)

    base = NGRP * GRPC
    for u in range(B):
        step(base + u, u, True)
    for u in range(B, QB):
        step(base + u, u, False)

    plsc.subcore_barrier()
    pltpu.sync_copy(acc_sh.at[pl.ds(r0, RPT)], sum_hbm.at[c, pl.ds(r0, RPT)])


def _sc_cnt_body(ip_hbm, z_hbm, ones_hbm, cnt_hbm,
                 dsti_v, ones_v, acc_sh, sem):
    c = lax.axis_index("c")
    s = lax.axis_index("s")
    wid = c * NS + s
    r0 = pl.multiple_of(s * RPT, 8)

    pltpu.sync_copy(z_hbm.at[pl.ds(r0, RPT)], acc_sh.at[pl.ds(r0, RPT)])
    pltpu.sync_copy(ones_hbm, ones_v)
    pltpu.sync_copy(ip_hbm.at[wid], dsti_v)
    plsc.subcore_barrier()

    @pl.loop(0, NCH, step=KC)
    def _(k0):
        for i in range(KC):
            pltpu.async_copy(ones_v, acc_sh.at[dsti_v.at[k0 + i, 1]], sem,
                             add=True)
        for i in range(KC):
            pltpu.make_async_copy(ones_v, acc_sh.at[dsti_v.at[k0 + i, 1]],
                                  sem).wait()

    plsc.subcore_barrier()
    pltpu.sync_copy(acc_sh.at[pl.ds(r0, RPT)], cnt_hbm.at[c, pl.ds(r0, RPT)])


_SC_MESH = plsc.VectorSubcoreMesh(core_axis_name="c", subcore_axis_name="s")

_sc_agg = pl.kernel(
    _sc_agg_body,
    out_type=jax.ShapeDtypeStruct((NC, NP, D), jnp.float32),
    mesh=_SC_MESH,
    scratch_types=[
        pltpu.VMEM((QB, 2, CH), jnp.int32),
        pltpu.VMEM((B, CH, D), jnp.float32),
        pltpu.VMEM_SHARED((NP, D), jnp.float32),
        pltpu.SemaphoreType.DMA,
        pltpu.SemaphoreType.DMA,
        pltpu.SemaphoreType.DMA,
        pltpu.SemaphoreType.DMA,
        pltpu.SemaphoreType.DMA,
        pltpu.SemaphoreType.DMA,
        pltpu.SemaphoreType.DMA,
        pltpu.SemaphoreType.DMA,
    ],
)

_sc_cnt = pl.kernel(
    _sc_cnt_body,
    out_type=jax.ShapeDtypeStruct((NC, NP, D), jnp.float32),
    mesh=_SC_MESH,
    scratch_types=[
        pltpu.VMEM((NCH, 2, CH), jnp.int32),
        pltpu.VMEM((CH, D), jnp.float32),
        pltpu.VMEM_SHARED((NP, D), jnp.float32),
        pltpu.SemaphoreType.DMA,
    ],
)


BLK = 2000


def _tc_body(final, sa, sb, ca, cb, x, wl, wr, b, o):
    cnt = jnp.maximum(ca[:, 0:1] + cb[:, 0:1], 1.0)
    aggr = (sa[:, :] + sb[:, :]) / cnt
    h = (jnp.dot(aggr, wl[:, :], preferred_element_type=jnp.float32)
         + jnp.dot(x[:, :], wr[:, :], preferred_element_type=jnp.float32)
         + b[:, :])
    if final:
        m = jnp.max(h, axis=1, keepdims=True)
        lse = jnp.log(jnp.sum(jnp.exp(h - m), axis=1, keepdims=True)) + m
        o[:, :] = h - lse
    else:
        o[:, :] = jnp.maximum(h, 0.0)


def _make_tc(final):
    row = pl.BlockSpec((BLK, D), lambda i: (i, 0))
    cntspec = pl.BlockSpec((BLK, CW), lambda i: (i, 0))
    full = pl.BlockSpec((D, D), lambda i: (0, 0))
    bias = pl.BlockSpec((1, D), lambda i: (0, 0))
    return pl.pallas_call(
        functools.partial(_tc_body, final),
        grid=(N // BLK,),
        in_specs=[row, row, cntspec, cntspec, row, full, full, bias],
        out_specs=row,
        out_shape=jax.ShapeDtypeStruct((N, D), jnp.float32),
    )


_tc_relu = _make_tc(False)
_tc_logsm = _make_tc(True)


def _pad_edges(src, dst):
    wid = jnp.arange(NW, dtype=jnp.int32)[:, None]
    pad_i = jnp.arange(PAD, dtype=jnp.int32)[None, :]
    pad_src = jnp.broadcast_to((pad_i * 89) % N, (NW, PAD))
    pad_dst = N + (wid * 37 + pad_i) % (NP - N)
    srcp = jnp.concatenate([src.reshape(NW, EPW), pad_src], axis=1)
    dstp = jnp.concatenate([dst.reshape(NW, EPW), pad_dst], axis=1)
    return jnp.stack([srcp.reshape(NW, NCH, CH),
                      dstp.reshape(NW, NCH, CH)], axis=2)


def kernel(x, edge_index, W1_l, b1, W1_r, W2_l, b2, W2_r):
    src = edge_index[0].astype(jnp.int32)
    dst = edge_index[1].astype(jnp.int32)
    ip = _pad_edges(src, dst)
    zeros = jnp.zeros((NP, D), jnp.float32)
    ones = jnp.ones((CH, D), jnp.float32)

    cnt = _sc_cnt(ip, zeros, ones)
    ca = cnt[0, :N, :CW]
    cb = cnt[1, :N, :CW]
    sum1 = _sc_agg(x, ip, zeros)
    h = _tc_relu(sum1[0, :N], sum1[1, :N], ca, cb, x,
                 W1_l.T, W1_r.T, b1[None, :])
    sum2 = _sc_agg(h, ip, zeros)
    return _tc_logsm(sum2[0, :N], sum2[1, :N], ca, cb, h,
                     W2_l.T, W2_r.T, b2[None, :])


# speedup vs baseline: 9.5037x; 1.0334x over previous
import functools

import jax
import jax.numpy as jnp
from jax import lax
from jax.experimental import pallas as pl
from jax.experimental.pallas import tpu as pltpu
from jax.experimental.pallas import tpu_sc as plsc

N = 10000
E = 320000
D = 128

NC = 2
NS = 16
NW = NC * NS
EPW = E // NW
CH = 64
NCH = 160
EPWP = NCH * CH
PAD = EPWP - EPW
CHC = 128
NCHC = EPWP // CHC
RPT = 632
NP = RPT * NS
RB = 2000
B = 4
QB = 2 * B
GRPC = QB
NGRP = NCH // GRPC - 1
KC = 8


def _sc_agg_body(x_hbm, ip_hbm, z_hbm, sum_hbm, idx_v, rows_v, acc_sh,
                 g0, g1, g2, g3, i0, i1, i2, i3, s0, s1, s2, s3):
    gs = (g0, g1, g2, g3)
    isems = (i0, i1, i2, i3)
    ss = (s0, s1, s2, s3)
    c = lax.axis_index("c")
    s = lax.axis_index("s")
    wid = c * NS + s
    r0 = pl.multiple_of(s * RPT, 8)

    pltpu.sync_copy(z_hbm.at[pl.ds(r0, RPT)], acc_sh.at[pl.ds(r0, RPT)])
    plsc.subcore_barrier()

    for b in range(B):
        pltpu.sync_copy(ip_hbm.at[wid, b], idx_v.at[b])
        pltpu.async_copy(x_hbm.at[idx_v.at[b, 0]], rows_v.at[b], gs[b])

    def step(k, u, issue_next):
        b = u % B
        q = u % QB
        qn = (u + B) % QB
        pltpu.make_async_copy(
            x_hbm.at[idx_v.at[q, 0]], rows_v.at[b], gs[b]).wait()
        if issue_next:
            pltpu.async_copy(ip_hbm.at[wid, k + B], idx_v.at[qn], isems[b])
        pltpu.async_copy(rows_v.at[b], acc_sh.at[idx_v.at[q, 1]], ss[b],
                         add=True)
        if issue_next:
            pltpu.make_async_copy(
                ip_hbm.at[wid, k + B], idx_v.at[qn], isems[b]).wait()
            pltpu.make_async_copy(
                rows_v.at[b], acc_sh.at[idx_v.at[q, 1]], ss[b]).wait()
            pltpu.async_copy(x_hbm.at[idx_v.at[qn, 0]], rows_v.at[b], gs[b])
        else:
            pltpu.make_async_copy(
                rows_v.at[b], acc_sh.at[idx_v.at[q, 1]], ss[b]).wait()

    @pl.loop(0, NGRP)
    def _(g):
        for u in range(GRPC):
            step(g * GRPC + u, u, True)

    base = NGRP * GRPC
    for u in range(B):
        step(base + u, u, True)
    for u in range(B, QB):
        step(base + u, u, False)

    plsc.subcore_barrier()
    pltpu.sync_copy(acc_sh.at[pl.ds(r0, RPT)], sum_hbm.at[c, pl.ds(r0, RPT)])


def _sc_cnt_body(dstc_hbm, z_hbm, ones_hbm, cnt_hbm,
                 dsti_v, ones_v, acc_sh, sem):
    c = lax.axis_index("c")
    s = lax.axis_index("s")
    wid = c * NS + s
    r0 = pl.multiple_of(s * RPT, 8)

    pltpu.sync_copy(z_hbm.at[pl.ds(r0, RPT)], acc_sh.at[pl.ds(r0, RPT)])
    pltpu.sync_copy(ones_hbm, ones_v)
    pltpu.sync_copy(dstc_hbm.at[wid], dsti_v)
    plsc.subcore_barrier()

    @pl.loop(0, NCHC, step=KC)
    def _(k0):
        for i in range(KC):
            pltpu.async_copy(ones_v, acc_sh.at[dsti_v.at[k0 + i]], sem,
                             add=True)
        for i in range(KC):
            pltpu.make_async_copy(ones_v, acc_sh.at[dsti_v.at[k0 + i]],
                                  sem).wait()

    plsc.subcore_barrier()
    pltpu.sync_copy(acc_sh.at[pl.ds(r0, RPT)], cnt_hbm.at[c, pl.ds(r0, RPT)])


_SC_MESH = plsc.VectorSubcoreMesh(core_axis_name="c", subcore_axis_name="s")

_sc_agg = pl.kernel(
    _sc_agg_body,
    out_type=jax.ShapeDtypeStruct((NC, NP, D), jnp.float32),
    mesh=_SC_MESH,
    scratch_types=[
        pltpu.VMEM((QB, 2, CH), jnp.int32),
        pltpu.VMEM((B, CH, D), jnp.float32),
        pltpu.VMEM_SHARED((NP, D), jnp.float32),
    ] + [pltpu.SemaphoreType.DMA] * 12,
)

_sc_cnt = pl.kernel(
    _sc_cnt_body,
    out_type=jax.ShapeDtypeStruct((NC, NP, D), jnp.float32),
    mesh=_SC_MESH,
    scratch_types=[
        pltpu.VMEM((NCHC, CHC), jnp.int32),
        pltpu.VMEM((CHC, D), jnp.float32),
        pltpu.VMEM_SHARED((NP, D), jnp.float32),
        pltpu.SemaphoreType.DMA,
    ],
)


def _tc_body(final, sa, sb, ca, cb, x, wl, wr, b, o):
    cnt = jnp.maximum(ca[0][:, 0:1] + cb[0][:, 0:1], 1.0)
    aggr = (sa[0] + sb[0]) / cnt
    h = (jnp.dot(aggr, wl[:, :], preferred_element_type=jnp.float32)
         + jnp.dot(x[:, :], wr[:, :], preferred_element_type=jnp.float32)
         + b[:, :])
    if final:
        m = jnp.max(h, axis=1, keepdims=True)
        lse = jnp.log(jnp.sum(jnp.exp(h - m), axis=1, keepdims=True)) + m
        o[:, :] = h - lse
    else:
        o[:, :] = jnp.maximum(h, 0.0)


def _make_tc(final):
    parta = pl.BlockSpec((1, RB, D), lambda i: (0, i, 0))
    partb = pl.BlockSpec((1, RB, D), lambda i: (1, i, 0))
    row = pl.BlockSpec((RB, D), lambda i: (i, 0))
    full = pl.BlockSpec((D, D), lambda i: (0, 0))
    bias = pl.BlockSpec((1, D), lambda i: (0, 0))
    return pl.pallas_call(
        functools.partial(_tc_body, final),
        grid=(N // RB,),
        in_specs=[parta, partb, parta, partb, row, full, full, bias],
        out_specs=row,
        out_shape=jax.ShapeDtypeStruct((N, D), jnp.float32),
    )


_tc_relu = _make_tc(False)
_tc_logsm = _make_tc(True)


def _pad_edges(src, dst):
    wid = jnp.arange(NW, dtype=jnp.int32)[:, None]
    pad_i = jnp.arange(PAD, dtype=jnp.int32)[None, :]
    pad_src = jnp.broadcast_to((pad_i * 89) % N, (NW, PAD))
    pad_dst = N + (wid * 37 + pad_i) % (NP - N)
    srcp = jnp.concatenate([src.reshape(NW, EPW), pad_src], axis=1)
    dstp = jnp.concatenate([dst.reshape(NW, EPW), pad_dst], axis=1)
    ip = jnp.stack([srcp.reshape(NW, NCH, CH),
                    dstp.reshape(NW, NCH, CH)], axis=2)
    return ip, dstp.reshape(NW, NCHC, CHC)


def kernel(x, edge_index, W1_l, b1, W1_r, W2_l, b2, W2_r):
    src = edge_index[0].astype(jnp.int32)
    dst = edge_index[1].astype(jnp.int32)
    ip, dstc = _pad_edges(src, dst)
    zeros = jnp.zeros((NP, D), jnp.float32)
    ones = jnp.ones((CHC, D), jnp.float32)

    cnt = _sc_cnt(dstc, zeros, ones)
    sum1 = _sc_agg(x, ip, zeros)
    h = _tc_relu(sum1, sum1, cnt, cnt, x, W1_l.T, W1_r.T, b1[None, :])
    sum2 = _sc_agg(h, ip, zeros)
    return _tc_logsm(sum2, sum2, cnt, cnt, h, W2_l.T, W2_r.T, b2[None, :])


# speedup vs baseline: 11.9989x; 1.2625x over previous
import dataclasses
import functools

import jax
import jax.numpy as jnp
from jax import lax
from jax.experimental import pallas as pl
from jax.experimental.pallas import tpu as pltpu
from jax.experimental.pallas import tpu_sc as plsc

N = 10000
E = 320000
D = 128

NC = 2
NS = 16
NW = NC * NS
EPW = E // NW
CH = 64
NCH = 160
EPWP = NCH * CH
PAD = EPWP - EPW
CHC = 128
NCHC = EPWP // CHC
RPT = 632
NP = RPT * NS
RB = 2000
B = 4
QB = 2 * B
GRPC = QB
NGRP = NCH // GRPC - 1
KC = 8


def _sc_agg_body(x_hbm, ip_hbm, z_hbm, sum_hbm, idx_v, rows_v, acc_sh,
                 g0, g1, g2, g3, i0, i1, i2, i3, s0, s1, s2, s3):
    gs = (g0, g1, g2, g3)
    isems = (i0, i1, i2, i3)
    ss = (s0, s1, s2, s3)
    c = lax.axis_index("c")
    s = lax.axis_index("s")
    wid = c * NS + s
    r0 = pl.multiple_of(s * RPT, 8)

    pltpu.sync_copy(z_hbm.at[pl.ds(r0, RPT)], acc_sh.at[pl.ds(r0, RPT)])
    plsc.subcore_barrier()

    for b in range(B):
        pltpu.sync_copy(ip_hbm.at[wid, b], idx_v.at[b])
        pltpu.async_copy(x_hbm.at[idx_v.at[b, 0]], rows_v.at[b], gs[b])

    def step(k, u, issue_next):
        b = u % B
        q = u % QB
        qn = (u + B) % QB
        pltpu.make_async_copy(
            x_hbm.at[idx_v.at[q, 0]], rows_v.at[b], gs[b]).wait()
        if issue_next:
            pltpu.async_copy(ip_hbm.at[wid, k + B], idx_v.at[qn], isems[b])
        pltpu.async_copy(rows_v.at[b], acc_sh.at[idx_v.at[q, 1]], ss[b],
                         add=True)
        if issue_next:
            pltpu.make_async_copy(
                ip_hbm.at[wid, k + B], idx_v.at[qn], isems[b]).wait()
            pltpu.make_async_copy(
                rows_v.at[b], acc_sh.at[idx_v.at[q, 1]], ss[b]).wait()
            pltpu.async_copy(x_hbm.at[idx_v.at[qn, 0]], rows_v.at[b], gs[b])
        else:
            pltpu.make_async_copy(
                rows_v.at[b], acc_sh.at[idx_v.at[q, 1]], ss[b]).wait()

    @pl.loop(0, NGRP)
    def _(g):
        for u in range(GRPC):
            step(g * GRPC + u, u, True)

    base = NGRP * GRPC
    for u in range(B):
        step(base + u, u, True)
    for u in range(B, QB):
        step(base + u, u, False)

    plsc.subcore_barrier()
    pltpu.sync_copy(acc_sh.at[pl.ds(r0, RPT)], sum_hbm.at[c, pl.ds(r0, RPT)])


def _sc_cnt_body(dstc_hbm, z1_hbm, cnt_hbm, dsti_v, hist_v):
    c = lax.axis_index("c")
    s = lax.axis_index("s")
    wid = c * NS + s

    pltpu.sync_copy(dstc_hbm.at[wid], dsti_v)
    pltpu.sync_copy(z1_hbm, hist_v)
    ones16 = jnp.ones((16,), jnp.float32)

    @pl.loop(0, EPWP, step=16)
    def _(i):
        dv = dsti_v[pl.ds(i, 16)]
        plsc.addupdate_scatter(hist_v, [dv], ones16)

    pltpu.sync_copy(hist_v, cnt_hbm.at[wid])


_SC_MESH = plsc.VectorSubcoreMesh(core_axis_name="c", subcore_axis_name="s")

_sc_agg = pl.kernel(
    _sc_agg_body,
    out_type=jax.ShapeDtypeStruct((NC, NP, D), jnp.float32),
    mesh=_SC_MESH,
    scratch_types=[
        pltpu.VMEM((QB, 2, CH), jnp.int32),
        pltpu.VMEM((B, CH, D), jnp.float32),
        pltpu.VMEM_SHARED((NP, D), jnp.float32),
    ] + [pltpu.SemaphoreType.DMA] * 12,
)

_SC_CNT_PARAMS = pltpu.CompilerParams()
if "needs_layout_passes" in pltpu.CompilerParams.__dataclass_fields__:
    _SC_CNT_PARAMS = dataclasses.replace(
        _SC_CNT_PARAMS, needs_layout_passes=False)

_sc_cnt = pl.kernel(
    _sc_cnt_body,
    out_type=jax.ShapeDtypeStruct((NW, NP), jnp.float32),
    mesh=_SC_MESH,
    scratch_types=[
        pltpu.VMEM((EPWP,), jnp.int32),
        pltpu.VMEM((NP,), jnp.float32),
    ],
    compiler_params=_SC_CNT_PARAMS,
)


def _tc_body(final, sa, sb, ch, x, wl, wr, b, o):
    cnt = jnp.maximum(jnp.sum(ch[:, :], axis=1), 1.0)[:, None]
    aggr = (sa[0] + sb[0]) / cnt
    h = (jnp.dot(aggr, wl[:, :], preferred_element_type=jnp.float32)
         + jnp.dot(x[:, :], wr[:, :], preferred_element_type=jnp.float32)
         + b[:, :])
    if final:
        m = jnp.max(h, axis=1, keepdims=True)
        lse = jnp.log(jnp.sum(jnp.exp(h - m), axis=1, keepdims=True)) + m
        o[:, :] = h - lse
    else:
        o[:, :] = jnp.maximum(h, 0.0)


def _make_tc(final):
    parta = pl.BlockSpec((1, RB, D), lambda i: (0, i, 0))
    partb = pl.BlockSpec((1, RB, D), lambda i: (1, i, 0))
    cnts = pl.BlockSpec((RB, NW), lambda i: (i, 0))
    row = pl.BlockSpec((RB, D), lambda i: (i, 0))
    full = pl.BlockSpec((D, D), lambda i: (0, 0))
    bias = pl.BlockSpec((1, D), lambda i: (0, 0))
    return pl.pallas_call(
        functools.partial(_tc_body, final),
        grid=(N // RB,),
        in_specs=[parta, partb, cnts, row, full, full, bias],
        out_specs=row,
        out_shape=jax.ShapeDtypeStruct((N, D), jnp.float32),
    )


_tc_relu = _make_tc(False)
_tc_logsm = _make_tc(True)


def _pad_edges(src, dst):
    wid = jnp.arange(NW, dtype=jnp.int32)[:, None]
    pad_i = jnp.arange(PAD, dtype=jnp.int32)[None, :]
    pad_src = jnp.broadcast_to((pad_i * 89) % N, (NW, PAD))
    pad_dst = N + (wid * 37 + pad_i) % (NP - N)
    srcp = jnp.concatenate([src.reshape(NW, EPW), pad_src], axis=1)
    dstp = jnp.concatenate([dst.reshape(NW, EPW), pad_dst], axis=1)
    ip = jnp.stack([srcp.reshape(NW, NCH, CH),
                    dstp.reshape(NW, NCH, CH)], axis=2)
    return ip, dstp


def kernel(x, edge_index, W1_l, b1, W1_r, W2_l, b2, W2_r):
    src = edge_index[0].astype(jnp.int32)
    dst = edge_index[1].astype(jnp.int32)
    ip, dstc = _pad_edges(src, dst)
    zeros = jnp.zeros((NP, D), jnp.float32)

    cnt = _sc_cnt(dstc, jnp.zeros((NP,), jnp.float32)).T
    sum1 = _sc_agg(x, ip, zeros)
    h = _tc_relu(sum1, sum1, cnt, x, W1_l.T, W1_r.T, b1[None, :])
    sum2 = _sc_agg(h, ip, zeros)
    return _tc_logsm(sum2, sum2, cnt, h, W2_l.T, W2_r.T, b2[None, :])


# speedup vs baseline: 12.4406x; 1.0368x over previous
import dataclasses
import functools

import jax
import jax.numpy as jnp
from jax import lax
from jax.experimental import pallas as pl
from jax.experimental.pallas import tpu as pltpu
from jax.experimental.pallas import tpu_sc as plsc

N = 10000
E = 320000
D = 128

NC = 2
NS = 16
NW = NC * NS
EPW = E // NW
CH = 128
NCH = 80
EPWP = NCH * CH
PAD = EPWP - EPW
RPT = 632
NP = RPT * NS
RB = 2000
B = 2
QB = 2 * B
GRPC = QB
NGRP = NCH // GRPC - 1


def _sc_agg_body(x_hbm, ip_hbm, z_hbm, sum_hbm, idx_v, rows_v, acc_sh,
                 *sems):
    gs = sems[:B]
    isems = sems[B:2 * B]
    ss = sems[2 * B:]
    c = lax.axis_index("c")
    s = lax.axis_index("s")
    wid = c * NS + s
    r0 = pl.multiple_of(s * RPT, 8)

    pltpu.sync_copy(z_hbm.at[pl.ds(r0, RPT)], acc_sh.at[pl.ds(r0, RPT)])
    plsc.subcore_barrier()

    for b in range(B):
        pltpu.sync_copy(ip_hbm.at[wid, b], idx_v.at[b])
        pltpu.async_copy(x_hbm.at[idx_v.at[b, 0]], rows_v.at[b], gs[b])

    def step(k, u, issue_next):
        b = u % B
        q = u % QB
        qn = (u + B) % QB
        pltpu.make_async_copy(
            x_hbm.at[idx_v.at[q, 0]], rows_v.at[b], gs[b]).wait()
        if issue_next:
            pltpu.async_copy(ip_hbm.at[wid, k + B], idx_v.at[qn], isems[b])
        pltpu.async_copy(rows_v.at[b], acc_sh.at[idx_v.at[q, 1]], ss[b],
                         add=True)
        if issue_next:
            pltpu.make_async_copy(
                ip_hbm.at[wid, k + B], idx_v.at[qn], isems[b]).wait()
            pltpu.make_async_copy(
                rows_v.at[b], acc_sh.at[idx_v.at[q, 1]], ss[b]).wait()
            pltpu.async_copy(x_hbm.at[idx_v.at[qn, 0]], rows_v.at[b], gs[b])
        else:
            pltpu.make_async_copy(
                rows_v.at[b], acc_sh.at[idx_v.at[q, 1]], ss[b]).wait()

    @pl.loop(0, NGRP)
    def _(g):
        for u in range(GRPC):
            step(g * GRPC + u, u, True)

    base = NGRP * GRPC
    for u in range(B):
        step(base + u, u, True)
    for u in range(B, QB):
        step(base + u, u, False)

    plsc.subcore_barrier()
    pltpu.sync_copy(acc_sh.at[pl.ds(r0, RPT)], sum_hbm.at[c, pl.ds(r0, RPT)])


def _sc_cnt_body(dstc_hbm, z1_hbm, cnt_hbm, dsti_v, hist_v):
    c = lax.axis_index("c")
    s = lax.axis_index("s")
    wid = c * NS + s

    pltpu.sync_copy(dstc_hbm.at[wid], dsti_v)
    pltpu.sync_copy(z1_hbm, hist_v)
    ones16 = jnp.ones((16,), jnp.float32)

    @pl.loop(0, EPWP, step=16)
    def _(i):
        dv = dsti_v[pl.ds(i, 16)]
        plsc.addupdate_scatter(hist_v, [dv], ones16)

    pltpu.sync_copy(hist_v, cnt_hbm.at[wid])


_SC_MESH = plsc.VectorSubcoreMesh(core_axis_name="c", subcore_axis_name="s")

_sc_agg = pl.kernel(
    _sc_agg_body,
    out_type=jax.ShapeDtypeStruct((NC, NP, D), jnp.float32),
    mesh=_SC_MESH,
    scratch_types=[
        pltpu.VMEM((QB, 2, CH), jnp.int32),
        pltpu.VMEM((B, CH, D), jnp.float32),
        pltpu.VMEM_SHARED((NP, D), jnp.float32),
    ] + [pltpu.SemaphoreType.DMA] * (3 * B),
)

_SC_CNT_PARAMS = pltpu.CompilerParams()
if "needs_layout_passes" in pltpu.CompilerParams.__dataclass_fields__:
    _SC_CNT_PARAMS = dataclasses.replace(
        _SC_CNT_PARAMS, needs_layout_passes=False)

_sc_cnt = pl.kernel(
    _sc_cnt_body,
    out_type=jax.ShapeDtypeStruct((NW, NP), jnp.float32),
    mesh=_SC_MESH,
    scratch_types=[
        pltpu.VMEM((EPWP,), jnp.int32),
        pltpu.VMEM((NP,), jnp.float32),
    ],
    compiler_params=_SC_CNT_PARAMS,
)


def _tc_body(final, sa, sb, ch, x, wl, wr, b, o):
    cnt = jnp.maximum(jnp.sum(ch[:, :], axis=1), 1.0)[:, None]
    aggr = (sa[0] + sb[0]) / cnt
    h = (jnp.dot(aggr, wl[:, :], preferred_element_type=jnp.float32)
         + jnp.dot(x[:, :], wr[:, :], preferred_element_type=jnp.float32)
         + b[:, :])
    if final:
        m = jnp.max(h, axis=1, keepdims=True)
        lse = jnp.log(jnp.sum(jnp.exp(h - m), axis=1, keepdims=True)) + m
        o[:, :] = h - lse
    else:
        o[:, :] = jnp.maximum(h, 0.0)


def _make_tc(final):
    parta = pl.BlockSpec((1, RB, D), lambda i: (0, i, 0))
    partb = pl.BlockSpec((1, RB, D), lambda i: (1, i, 0))
    cnts = pl.BlockSpec((RB, NW), lambda i: (i, 0))
    row = pl.BlockSpec((RB, D), lambda i: (i, 0))
    full = pl.BlockSpec((D, D), lambda i: (0, 0))
    bias = pl.BlockSpec((1, D), lambda i: (0, 0))
    return pl.pallas_call(
        functools.partial(_tc_body, final),
        grid=(N // RB,),
        in_specs=[parta, partb, cnts, row, full, full, bias],
        out_specs=row,
        out_shape=jax.ShapeDtypeStruct((N, D), jnp.float32),
    )


_tc_relu = _make_tc(False)
_tc_logsm = _make_tc(True)


def _pad_edges(src, dst):
    wid = jnp.arange(NW, dtype=jnp.int32)[:, None]
    pad_i = jnp.arange(PAD, dtype=jnp.int32)[None, :]
    pad_src = jnp.broadcast_to((pad_i * 89) % N, (NW, PAD))
    pad_dst = N + (wid * 37 + pad_i) % (NP - N)
    srcp = jnp.concatenate([src.reshape(NW, EPW), pad_src], axis=1)
    dstp = jnp.concatenate([dst.reshape(NW, EPW), pad_dst], axis=1)
    ip = jnp.stack([srcp.reshape(NW, NCH, CH),
                    dstp.reshape(NW, NCH, CH)], axis=2)
    return ip, dstp


def kernel(x, edge_index, W1_l, b1, W1_r, W2_l, b2, W2_r):
    src = edge_index[0].astype(jnp.int32)
    dst = edge_index[1].astype(jnp.int32)
    ip, dstc = _pad_edges(src, dst)
    zeros = jnp.zeros((NP, D), jnp.float32)

    cnt = _sc_cnt(dstc, jnp.zeros((NP,), jnp.float32)).T
    sum1 = _sc_agg(x, ip, zeros)
    h = _tc_relu(sum1, sum1, cnt, x, W1_l.T, W1_r.T, b1[None, :])
    sum2 = _sc_agg(h, ip, zeros)
    return _tc_logsm(sum2, sum2, cnt, h, W2_l.T, W2_r.T, b2[None, :])
